# Initial kernel scaffold; baseline (speedup 1.0000x reference)
import functools

import jax
import jax.numpy as jnp
from jax import lax
from jax.experimental import pallas as pl
from jax.experimental.pallas import tpu as pltpu
from jax.experimental.pallas import tpu_sc as plsc

N_CLASS = 1000
BATCH = 16384
NC, NS = 2, 16
NW = NC * NS
ROWS_PER_W = BATCH // NW
CR = 8
NCHUNK = ROWS_PER_W // CR
ZCH = CR * N_CLASS
OCH = CR * 2 * N_CLASS
VPR = 2 * N_CLASS // 16


@functools.partial(
    pl.kernel,
    out_type=jax.ShapeDtypeStruct((BATCH * 2 * N_CLASS,), jnp.float32),
    mesh=plsc.VectorSubcoreMesh(core_axis_name="c", subcore_axis_name="s"),
    scratch_types=[
        pltpu.VMEM((ZCH,), jnp.int32),
        pltpu.VMEM((OCH,), jnp.float32),
        pltpu.VMEM((2 * N_CLASS,), jnp.float32),
        pltpu.VMEM((2 * N_CLASS,), jnp.float32),
    ],
)
def _zy_sc(z_hbm, u_hbm, v_hbm, out_hbm, z_v, out_v, u_v, v_v):
    wid = lax.axis_index("c") * NS + lax.axis_index("s")
    base = wid * ROWS_PER_W * N_CLASS
    obase = 2 * base

    pltpu.sync_copy(u_hbm, u_v)
    pltpu.sync_copy(v_hbm, v_v)

    half_iota = lax.iota(jnp.int32, 16) >> 1

    def chunk(ch, _):
        zoff = base + ch * ZCH
        ooff = obase + ch * OCH
        pltpu.sync_copy(z_hbm.at[pl.ds(zoff, ZCH)], z_v)

        def col(j, _):
            uf = u_v[pl.ds(j * 16, 16)]
            vf = v_v[pl.ds(j * 16, 16)]
            for r in range(CR):
                idx = half_iota + (r * N_CLASS + j * 8)
                zf = plsc.load_gather(z_v, [idx]).astype(jnp.float32)
                out_v[pl.ds(r * 2 * N_CLASS + j * 16, 16)] = zf * uf + vf
            return 0

        lax.fori_loop(0, VPR, col, 0)
        pltpu.sync_copy(out_v, out_hbm.at[pl.ds(ooff, OCH)])
        return 0

    lax.fori_loop(0, NCHUNK, chunk, 0)


def kernel(z, phi, mask):
    pf = phi.reshape(-1)
    a = mask[1, 0] - mask[0, 0] - mask[1, 1] + mask[0, 1]
    b = mask[1, 1] - mask[0, 1]
    c = mask[0, 0] - mask[0, 1]
    d = mask[0, 1]
    u = a * pf + b
    v = c * pf + d
    zf = z.reshape(-1).astype(jnp.int32)
    out = _zy_sc(zf, u, v)
    return out.reshape(BATCH, N_CLASS, 2)


# speedup vs baseline: 5.3870x; 5.3870x over previous
import functools

import jax
import jax.numpy as jnp
from jax import lax
from jax.experimental import pallas as pl
from jax.experimental.pallas import tpu as pltpu
from jax.experimental.pallas import tpu_sc as plsc

N_CLASS = 1000
BATCH = 16384
NC, NS = 2, 16
NW = NC * NS
ROWS_PER_W = BATCH // NW
CR = 8
NCHUNK = ROWS_PER_W // CR
ZCH = CR * N_CLASS
OCH = CR * 2 * N_CLASS
VPR = 2 * N_CLASS // 16


@functools.partial(
    pl.kernel,
    out_type=jax.ShapeDtypeStruct((BATCH * 2 * N_CLASS,), jnp.float32),
    mesh=plsc.VectorSubcoreMesh(core_axis_name="c", subcore_axis_name="s"),
    compiler_params=pltpu.CompilerParams(needs_layout_passes=False),
    scratch_types=[
        pltpu.VMEM((ZCH,), jnp.int32),
        pltpu.VMEM((OCH,), jnp.float32),
        pltpu.VMEM((2 * N_CLASS,), jnp.float32),
        pltpu.VMEM((2 * N_CLASS,), jnp.float32),
    ],
)
def _zy_sc(z_hbm, u_hbm, v_hbm, out_hbm, z_v, out_v, u_v, v_v):
    wid = lax.axis_index("c") * NS + lax.axis_index("s")
    base = wid * ROWS_PER_W * N_CLASS
    obase = 2 * base

    pltpu.sync_copy(u_hbm, u_v)
    pltpu.sync_copy(v_hbm, v_v)

    half_iota = lax.iota(jnp.int32, 16) >> 1

    def chunk(ch, _):
        zoff = base + ch * ZCH
        ooff = obase + ch * OCH
        pltpu.sync_copy(z_hbm.at[pl.ds(zoff, ZCH)], z_v)

        def col(j, _):
            uf = u_v[pl.ds(j * 16, 16)]
            vf = v_v[pl.ds(j * 16, 16)]
            for r in range(CR):
                idx = half_iota + (r * N_CLASS + j * 8)
                zf = plsc.load_gather(z_v, [idx]).astype(jnp.float32)
                out_v[pl.ds(r * 2 * N_CLASS + j * 16, 16)] = zf * uf + vf
            return 0

        lax.fori_loop(0, VPR, col, 0)
        pltpu.sync_copy(out_v, out_hbm.at[pl.ds(ooff, OCH)])
        return 0

    lax.fori_loop(0, NCHUNK, chunk, 0)


def kernel(z, phi, mask):
    pf = phi.reshape(-1)
    a = mask[1, 0] - mask[0, 0] - mask[1, 1] + mask[0, 1]
    b = mask[1, 1] - mask[0, 1]
    c = mask[0, 0] - mask[0, 1]
    d = mask[0, 1]
    u = a * pf + b
    v = c * pf + d
    zf = z.reshape(-1).astype(jnp.int32)
    out = _zy_sc(zf, u, v)
    return out.reshape(BATCH, N_CLASS, 2)


# speedup vs baseline: 5.6315x; 1.0454x over previous
import functools

import jax
import jax.numpy as jnp
from jax import lax
from jax.experimental import pallas as pl
from jax.experimental.pallas import tpu as pltpu
from jax.experimental.pallas import tpu_sc as plsc

N_CLASS = 1000
BATCH = 16384
NC, NS = 2, 16
NW = NC * NS
ROWS_PER_W = BATCH // NW
CR = 16
NCHUNK = ROWS_PER_W // CR
ZCH = CR * N_CLASS
OCH = CR * 2 * N_CLASS
VPR = 2 * N_CLASS // 16


@functools.partial(
    pl.kernel,
    out_type=jax.ShapeDtypeStruct((BATCH * 2 * N_CLASS,), jnp.float32),
    mesh=plsc.VectorSubcoreMesh(core_axis_name="c", subcore_axis_name="s"),
    compiler_params=pltpu.CompilerParams(
        needs_layout_passes=False, use_tc_tiling_on_sc=False
    ),
    scratch_types=[
        pltpu.VMEM((ZCH,), jnp.int32),
        pltpu.VMEM((ZCH,), jnp.int32),
        pltpu.VMEM((OCH,), jnp.float32),
        pltpu.VMEM((OCH,), jnp.float32),
        pltpu.VMEM((2 * N_CLASS,), jnp.float32),
        pltpu.VMEM((2 * N_CLASS,), jnp.float32),
        pltpu.SemaphoreType.DMA,
        pltpu.SemaphoreType.DMA,
        pltpu.SemaphoreType.DMA,
        pltpu.SemaphoreType.DMA,
    ],
)
def _zy_sc(z_hbm, u_hbm, v_hbm, out_hbm,
           z_v0, z_v1, out_v0, out_v1, u_v, v_v,
           zsem0, zsem1, osem0, osem1):
    wid = lax.axis_index("c") * NS + lax.axis_index("s")
    base = wid * ROWS_PER_W * N_CLASS
    obase = 2 * base

    pltpu.sync_copy(u_hbm, u_v)
    pltpu.sync_copy(v_hbm, v_v)

    half_iota = lax.iota(jnp.int32, 16) >> 1
    zbufs = (z_v0, z_v1)
    obufs = (out_v0, out_v1)
    zsems = (zsem0, zsem1)
    osems = (osem0, osem1)

    def zslice(g):
        return z_hbm.at[pl.ds(base + g * ZCH, ZCH)]

    def oslice(g):
        return out_hbm.at[pl.ds(obase + g * OCH, OCH)]

    pltpu.async_copy(zslice(0), z_v0, zsem0)
    pltpu.async_copy(zslice(1), z_v1, zsem1)

    def pair(i, _):
        for b in range(2):
            g = 2 * i + b
            zv, ov = zbufs[b], obufs[b]
            zs, os = zsems[b], osems[b]
            pltpu.make_async_copy(zslice(g), zv, zs).wait()

            @pl.when(i > 0)
            def _():
                pltpu.make_async_copy(ov, oslice(g - 2), os).wait()

            @plsc.parallel_loop(0, VPR, 1)
            def col(j):
                uf = u_v[pl.ds(j * 16, 16)]
                vf = v_v[pl.ds(j * 16, 16)]
                zs_ = [
                    plsc.load_gather(zv, [half_iota + (r * N_CLASS + j * 8)])
                    for r in range(CR)
                ]
                for r in range(CR):
                    zf = zs_[r].astype(jnp.float32)
                    ov[pl.ds(r * 2 * N_CLASS + j * 16, 16)] = zf * uf + vf

            pltpu.async_copy(ov, oslice(g), os)

            @pl.when(g + 2 < NCHUNK)
            def _():
                pltpu.async_copy(zslice(g + 2), zv, zs)

        return 0

    lax.fori_loop(0, NCHUNK // 2, pair, 0)
    pltpu.make_async_copy(out_v0, oslice(NCHUNK - 2), osem0).wait()
    pltpu.make_async_copy(out_v1, oslice(NCHUNK - 1), osem1).wait()


def kernel(z, phi, mask):
    pf = phi.reshape(-1)
    a = mask[1, 0] - mask[0, 0] - mask[1, 1] + mask[0, 1]
    b = mask[1, 1] - mask[0, 1]
    c = mask[0, 0] - mask[0, 1]
    d = mask[0, 1]
    u = a * pf + b
    v = c * pf + d
    zf = z.reshape(-1).astype(jnp.int32)
    out = _zy_sc(zf, u, v)
    return out.reshape(BATCH, N_CLASS, 2)
